# baseline (speedup 1.0000x reference)
import jax
import jax.numpy as jnp
from jax import lax
from jax.experimental import pallas as pl
from jax.experimental.pallas import tpu as pltpu
from jax.experimental.pallas import tpu_sc as plsc

_LENS = (480000, 448000, 416000, 384000, 352000, 320000, 288000, 256000)
_MAXL = 480000
_NC, _NS = 2, 16
_NW = _NC * _NS
_W = 3200
_NT = _MAXL // _W
_TPC = 32000 // _W
_NSLOT = -(-_NT // _NW)


def _body(w0, w1, w2, w3, w4, w5, w6, w7, zrow, out, buf, *sems):
    ws = (w0, w1, w2, w3, w4, w5, w6, w7)
    isems, osems = sems[:2], sems[2:]
    wid = lax.axis_index("s") * _NC + lax.axis_index("c")

    def task(slot):
        return wid + slot * _NW

    def valid(slot):
        return True if (slot + 1) * _NW <= _NT else task(slot) < _NT

    def c0(slot):
        return pl.multiple_of(task(slot) * _W, _W)

    def in_copy(slot, r):
        return pltpu.make_async_copy(
            ws[r].at[pl.ds(c0(slot), _W)], buf.at[slot % 2, r], isems[slot % 2]
        )

    def zero_copy(slot, r):
        return pltpu.make_async_copy(zrow, buf.at[slot % 2, r], isems[slot % 2])

    def out_copy(slot):
        return pltpu.make_async_copy(
            buf.at[slot % 2], out.at[:, pl.ds(c0(slot), _W)], osems[slot % 2]
        )

    def in_start(slot):
        t = task(slot)
        v = valid(slot)
        for r in range(8):
            data = t < (15 - r) * _TPC

            @pl.when(v & data)
            def _(slot=slot, r=r):
                in_copy(slot, r).start()

            @pl.when(v & jnp.logical_not(data))
            def _(slot=slot, r=r):
                zero_copy(slot, r).start()

    def guarded(slot, f):
        if valid(slot) is True:
            f(slot)
        else:

            @pl.when(valid(slot))
            def _():
                f(slot)

    def in_wait(slot):
        guarded(slot, lambda s: [in_copy(s, r).wait() for r in range(8)])

    def out_start(slot):
        guarded(slot, lambda s: out_copy(s).start())

    def out_wait(slot):
        guarded(slot, lambda s: out_copy(s).wait())

    for slot in range(_NSLOT):
        if slot >= 2:
            out_wait(slot - 2)
        in_start(slot)
        if slot >= 1:
            in_wait(slot - 1)
            out_start(slot - 1)

    in_wait(_NSLOT - 1)
    out_start(_NSLOT - 1)
    if _NSLOT >= 2:
        out_wait(_NSLOT - 2)
    out_wait(_NSLOT - 1)


@jax.jit
def _pad_stack(w0, w1, w2, w3, w4, w5, w6, w7):
    mesh = plsc.VectorSubcoreMesh(core_axis_name="c", subcore_axis_name="s")
    f = pl.kernel(
        _body,
        out_type=jax.ShapeDtypeStruct((8, _MAXL), jnp.float32),
        mesh=mesh,
        scratch_types=[
            pltpu.VMEM((2, 8, _W), jnp.float32),
        ] + [pltpu.SemaphoreType.DMA] * 4,
    )
    zrow = jnp.zeros((_W,), jnp.float32)
    return f(w0, w1, w2, w3, w4, w5, w6, w7, zrow)


def kernel(w0, w1, w2, w3, w4, w5, w6, w7):
    batched = _pad_stack(w0, w1, w2, w3, w4, w5, w6, w7)
    wave_lengths = jnp.array(_LENS, dtype=jnp.int32)
    return (batched, wave_lengths)


# speedup vs baseline: 1.1273x; 1.1273x over previous
import jax
import jax.numpy as jnp
from jax import lax
from jax.experimental import pallas as pl
from jax.experimental.pallas import tpu as pltpu
from jax.experimental.pallas import tpu_sc as plsc

_LENS = (480000, 448000, 416000, 384000, 352000, 320000, 288000, 256000)
_MAXL = 480000
_NC, _NS = 2, 16
_NW = _NC * _NS
_W = 6400
_NT = _MAXL // _W
_TPC = 32000 // _W
_NSLOT = -(-_NT // _NW)


def _body(w0, w1, w2, w3, w4, w5, w6, w7, zrow, out, buf, *sems):
    ws = (w0, w1, w2, w3, w4, w5, w6, w7)
    isems, osems = sems[:2], sems[2:]
    wid = lax.axis_index("s") * _NC + lax.axis_index("c")

    def task(slot):
        return wid + slot * _NW

    def valid(slot):
        return True if (slot + 1) * _NW <= _NT else task(slot) < _NT

    def c0(slot):
        return pl.multiple_of(task(slot) * _W, _W)

    def in_copy(slot, r):
        return pltpu.make_async_copy(
            ws[r].at[pl.ds(c0(slot), _W)], buf.at[slot % 2, r], isems[slot % 2]
        )

    def zero_copy(slot, r):
        return pltpu.make_async_copy(zrow, buf.at[slot % 2, r], isems[slot % 2])

    def out_copy(slot):
        return pltpu.make_async_copy(
            buf.at[slot % 2], out.at[:, pl.ds(c0(slot), _W)], osems[slot % 2]
        )

    def in_start(slot):
        t = task(slot)
        v = valid(slot)
        for r in range(8):
            data = t < (15 - r) * _TPC

            @pl.when(v & data)
            def _(slot=slot, r=r):
                in_copy(slot, r).start()

            @pl.when(v & jnp.logical_not(data))
            def _(slot=slot, r=r):
                zero_copy(slot, r).start()

    def guarded(slot, f):
        if valid(slot) is True:
            f(slot)
        else:

            @pl.when(valid(slot))
            def _():
                f(slot)

    def in_wait(slot):
        guarded(slot, lambda s: [in_copy(s, r).wait() for r in range(8)])

    def out_start(slot):
        guarded(slot, lambda s: out_copy(s).start())

    def out_wait(slot):
        guarded(slot, lambda s: out_copy(s).wait())

    for slot in range(_NSLOT):
        if slot >= 2:
            out_wait(slot - 2)
        in_start(slot)
        if slot >= 1:
            in_wait(slot - 1)
            out_start(slot - 1)

    in_wait(_NSLOT - 1)
    out_start(_NSLOT - 1)
    if _NSLOT >= 2:
        out_wait(_NSLOT - 2)
    out_wait(_NSLOT - 1)


@jax.jit
def _pad_stack(w0, w1, w2, w3, w4, w5, w6, w7):
    mesh = plsc.VectorSubcoreMesh(core_axis_name="c", subcore_axis_name="s")
    f = pl.kernel(
        _body,
        out_type=jax.ShapeDtypeStruct((8, _MAXL), jnp.float32),
        mesh=mesh,
        scratch_types=[
            pltpu.VMEM((2, 8, _W), jnp.float32),
        ] + [pltpu.SemaphoreType.DMA] * 4,
    )
    zrow = jnp.zeros((_W,), jnp.float32)
    return f(w0, w1, w2, w3, w4, w5, w6, w7, zrow)


def kernel(w0, w1, w2, w3, w4, w5, w6, w7):
    batched = _pad_stack(w0, w1, w2, w3, w4, w5, w6, w7)
    wave_lengths = jnp.array(_LENS, dtype=jnp.int32)
    return (batched, wave_lengths)


# speedup vs baseline: 1.1690x; 1.0370x over previous
import jax
import jax.numpy as jnp
import numpy as np
from jax import lax
from jax.experimental import pallas as pl
from jax.experimental.pallas import tpu as pltpu
from jax.experimental.pallas import tpu_sc as plsc

_LENS = (480000, 448000, 416000, 384000, 352000, 320000, 288000, 256000)
_MAXL = 480000
_NC, _NS = 2, 16
_NW = _NC * _NS
_W = 16000
_NT = _MAXL // _W
_TPC = 32000 // _W
_H = (7936, 8064)
_HOFF = (0, 7936)


def _body(w0, w1, w2, w3, w4, w5, w6, w7, zrow, out, bufs, isems, osem):
    ws = (w0, w1, w2, w3, w4, w5, w6, w7)
    wid = lax.axis_index("s") * _NC + lax.axis_index("c")
    t = wid
    valid = t < _NT
    c0 = pl.multiple_of(t * _W, _W)

    def in_copy(h, r):
        return pltpu.make_async_copy(
            ws[r].at[pl.ds(c0 + _HOFF[h], _H[h])], bufs[h].at[r], isems[h]
        )

    def zero_copy(h, r):
        return pltpu.make_async_copy(
            zrow.at[pl.ds(0, _H[h])], bufs[h].at[r], isems[h]
        )

    def out_copy(h):
        return pltpu.make_async_copy(
            bufs[h], out.at[:, pl.ds(c0 + _HOFF[h], _H[h])], osem
        )

    for h in (0, 1):
        for r in range(8):
            data = t < (15 - r) * _TPC

            @pl.when(valid & data)
            def _(h=h, r=r):
                in_copy(h, r).start()

            @pl.when(valid & jnp.logical_not(data))
            def _(h=h, r=r):
                zero_copy(h, r).start()

    @pl.when(valid)
    def _():
        for h in (0, 1):
            for r in range(8):
                in_copy(h, r).wait()
            out_copy(h).start()
        for h in (0, 1):
            out_copy(h).wait()


@jax.jit
def _pad_stack(w0, w1, w2, w3, w4, w5, w6, w7):
    mesh = plsc.VectorSubcoreMesh(core_axis_name="c", subcore_axis_name="s")
    f = pl.kernel(
        _body,
        out_type=jax.ShapeDtypeStruct((8, _MAXL), jnp.float32),
        mesh=mesh,
        scratch_types=[
            (pltpu.VMEM((8, _H[0]), jnp.float32),
             pltpu.VMEM((8, _H[1]), jnp.float32)),
            (pltpu.SemaphoreType.DMA, pltpu.SemaphoreType.DMA),
            pltpu.SemaphoreType.DMA,
        ],
    )
    zrow = jnp.asarray(np.zeros((max(_H),), np.float32))
    return f(w0, w1, w2, w3, w4, w5, w6, w7, zrow)


def kernel(w0, w1, w2, w3, w4, w5, w6, w7):
    batched = _pad_stack(w0, w1, w2, w3, w4, w5, w6, w7)
    wave_lengths = jnp.array(_LENS, dtype=jnp.int32)
    return (batched, wave_lengths)


# speedup vs baseline: 1.2416x; 1.0621x over previous
import jax
import jax.numpy as jnp
from jax import lax
from jax.experimental import pallas as pl
from jax.experimental.pallas import tpu as pltpu
from jax.experimental.pallas import tpu_sc as plsc

_LENS = (480000, 448000, 416000, 384000, 352000, 320000, 288000, 256000)
_MAXL = 480000
_NC, _NS = 2, 16
_NW = _NC * _NS
_W = 16000
_NT = _MAXL // _W
_TPC = 32000 // _W


def _body(w0, w1, w2, w3, w4, w5, w6, w7, zrow, out, buf, isem, osem):
    ws = (w0, w1, w2, w3, w4, w5, w6, w7)
    wid = lax.axis_index("s") * _NC + lax.axis_index("c")
    t = wid
    valid = t < _NT
    c0 = pl.multiple_of(t * _W, _W)

    def in_copy(r):
        return pltpu.make_async_copy(
            ws[r].at[pl.ds(c0, _W)], buf.at[r], isem
        )

    def zero_copy(r):
        return pltpu.make_async_copy(zrow, buf.at[r], isem)

    for r in range(8):
        data = t < (15 - r) * _TPC

        @pl.when(valid & data)
        def _(r=r):
            in_copy(r).start()

        @pl.when(valid & jnp.logical_not(data))
        def _(r=r):
            zero_copy(r).start()

    @pl.when(valid)
    def _():
        for r in range(8):
            in_copy(r).wait()

    out_copy = pltpu.make_async_copy(
        buf, out.at[:, pl.ds(c0, _W)], osem
    )

    @pl.when(valid)
    def _():
        out_copy.start()
        out_copy.wait()


@jax.jit
def _pad_stack(w0, w1, w2, w3, w4, w5, w6, w7):
    mesh = plsc.VectorSubcoreMesh(core_axis_name="c", subcore_axis_name="s")
    f = pl.kernel(
        _body,
        out_type=jax.ShapeDtypeStruct((8, _MAXL), jnp.float32),
        mesh=mesh,
        scratch_types=[
            pltpu.VMEM((8, _W), jnp.float32),
            pltpu.SemaphoreType.DMA,
            pltpu.SemaphoreType.DMA,
        ],
    )
    zrow = jnp.zeros((_W,), jnp.float32)
    return f(w0, w1, w2, w3, w4, w5, w6, w7, zrow)


def kernel(w0, w1, w2, w3, w4, w5, w6, w7):
    batched = _pad_stack(w0, w1, w2, w3, w4, w5, w6, w7)
    wave_lengths = jnp.array(_LENS, dtype=jnp.int32)
    return (batched, wave_lengths)
